# baseline (speedup 1.0000x reference)
import jax
import jax.numpy as jnp
from jax import lax
from jax.experimental import pallas as pl
from jax.experimental.pallas import tpu as pltpu
from jax.experimental.pallas import tpu_sc as plsc

N = 50000
SDIM = 256
VDIM = 64
B = 64
EPS = 1e-6

NW = 32
CHV = 1664
NFULL = 30
TAIL0 = NFULL * CHV
TAILN = N - TAIL0
SSUB = 208
NSSUB = CHV // SSUB

_F32 = jnp.float32
_I32 = jnp.int32


NPAD = 50048


def _stats_body(s_hbm, vt_hbm, vtail_hbm, b_hbm, out_hbm, brow_hbm, outf_hbm,
                sv0, sv1, vv, vtv, bv, acc, loc, locf, sem_s0, sem_s1, sem_v,
                sem_b):
    wid = lax.axis_index("s") * 2 + lax.axis_index("c")
    iota16 = lax.iota(_I32, 16)
    zeros16 = jnp.zeros((16,), _F32)
    lane0 = iota16 == 0
    lanebase = iota16 * 64

    def _z(j, c):
        acc[pl.ds(j * 16, 16)] = zeros16
        return c

    lax.fori_loop(0, 256, _z, 0)

    def _clamp(segs):
        return jnp.minimum(jnp.maximum(segs, 0), B - 1)

    def _vgroup(goff):
        segs = _clamp(bv[pl.ds(goff * 16, 16)])
        v0 = vv[0, 0, pl.ds(goff * 16, 16)]
        v1 = vv[1, 0, pl.ds(goff * 16, 16)]
        v2 = vv[2, 0, pl.ds(goff * 16, 16)]
        vsq = v0 * v0 + v1 * v1 + v2 * v2
        plsc.addupdate_scatter(acc, [lanebase + segs + 3072], vsq)

    def _sgroup(sv, boff, row0):
        segs = _clamp(bv[pl.ds(boff, 16)])
        umax = lax.reduce_max(segs, (0,))
        umin = lax.reduce_min(segs, (0,))

        @pl.when(umax == umin)
        def _():
            cs = zeros16
            sq = zeros16
            for jj in range(16):
                x = sv[row0 + jj, pl.ds(0, 16)]
                cs = cs + x
                sq = sq + x * x
            idx = jnp.full((16,), umax, _I32)
            plsc.addupdate_scatter(acc, [idx], jnp.full((16,), 16.0, _F32),
                                   mask=lane0)
            plsc.addupdate_scatter(acc, [idx + 1024], cs, mask=lane0)
            plsc.addupdate_scatter(acc, [idx + 2048], sq, mask=lane0)

        @pl.when(umax != umin)
        def _():
            for jj in range(16):
                x = sv[row0 + jj, pl.ds(0, 16)]
                idx = jnp.full((16,), segs[jj], _I32)
                plsc.addupdate_scatter(acc, [idx], jnp.full((16,), 1.0, _F32),
                                       mask=lane0)
                plsc.addupdate_scatter(acc, [idx + 1024], x, mask=lane0)
                plsc.addupdate_scatter(acc, [idx + 2048], x * x, mask=lane0)

    @pl.when(wid < NFULL)
    def _():
        base = wid * CHV
        svs = (sv0, sv1)
        sems = (sem_s0, sem_s1)
        hb = pltpu.async_copy(b_hbm.at[pl.ds(base, CHV)], bv, sem_b)
        hv = pltpu.async_copy(
            vt_hbm.at[pl.ds(0, 3), pl.ds(0, 8), pl.ds(base, CHV)], vv, sem_v)

        def _start_s(k):
            return pltpu.async_copy(
                s_hbm.at[pl.ds(base + k * SSUB, SSUB), pl.ds(0, 128)],
                svs[k % 2], sems[k % 2])

        hs = _start_s(0)
        hb.wait()
        pltpu.sync_copy(bv, brow_hbm.at[0, pl.ds(base, CHV)])
        for k in range(NSSUB):
            hs_next = _start_s(k + 1) if k + 1 < NSSUB else None
            hs.wait()

            def _sg(j, c2, _k=k):
                _sgroup(svs[_k % 2], _k * SSUB + j * 16, j * 16)
                return c2

            lax.fori_loop(0, SSUB // 16, _sg, 0)
            hs = hs_next

        hv.wait()

        def _vg(g, c):
            _vgroup(g)
            return c

        lax.fori_loop(0, CHV // 16, _vg, 0)

    @pl.when(wid == NFULL)
    def _():
        pltpu.sync_copy(b_hbm.at[pl.ds(TAIL0, TAILN)], bv.at[pl.ds(0, TAILN)])
        pltpu.sync_copy(bv.at[pl.ds(0, 128)],
                        brow_hbm.at[0, pl.ds(TAIL0, 128)])
        pltpu.sync_copy(s_hbm.at[pl.ds(TAIL0, TAILN), pl.ds(0, 128)],
                        sv0.at[pl.ds(0, TAILN)])
        for j in range(TAILN // 16):
            _sgroup(sv0, j * 16, j * 16)

    @pl.when(wid == NFULL + 1)
    def _():
        pltpu.sync_copy(b_hbm.at[pl.ds(TAIL0, TAILN)], bv.at[pl.ds(0, TAILN)])
        pltpu.sync_copy(vtail_hbm, vtv)
        for g in range(TAILN // 16):
            segs = _clamp(bv[pl.ds(g * 16, 16)])
            v0 = vtv[0, pl.ds(g * 16, 16)]
            v1 = vtv[1, pl.ds(g * 16, 16)]
            v2 = vtv[2, pl.ds(g * 16, 16)]
            vsq = v0 * v0 + v1 * v1 + v2 * v2
            plsc.addupdate_scatter(acc, [lanebase + segs + 3072], vsq)

    for g in range(4):
        segv = g * 16 + iota16
        for f in range(4, 8):
            plsc.store_scatter(loc, [segv, jnp.full((16,), f, _I32)], zeros16)
            plsc.store_scatter(locf, [segv * 8 + f], zeros16)
        for f in range(4):
            tot = zeros16
            for l in range(16):
                tot = tot + acc[pl.ds(f * 1024 + l * 64 + g * 16, 16)]
            plsc.store_scatter(loc, [segv, jnp.full((16,), f, _I32)], tot)
            plsc.store_scatter(locf, [segv * 8 + f], tot)
    pltpu.sync_copy(loc, out_hbm.at[wid])
    pltpu.sync_copy(locf, outf_hbm.at[wid])


def _stats_call(s, vt, vtail, batch):
    mesh = plsc.VectorSubcoreMesh(core_axis_name="c", subcore_axis_name="s")
    k = pl.kernel(
        _stats_body,
        mesh=mesh,
        out_type=(jax.ShapeDtypeStruct((NW, B, 8), _F32),
                  jax.ShapeDtypeStruct((1, NPAD), _I32),
                  jax.ShapeDtypeStruct((NW, B * 8), _F32)),
        scratch_types=[
            pltpu.VMEM((SSUB, 128), _F32),
            pltpu.VMEM((SSUB, 128), _F32),
            pltpu.VMEM((3, 8, CHV), _F32),
            pltpu.VMEM((3, TAILN), _F32),
            pltpu.VMEM((CHV,), _I32),
            pltpu.VMEM((16 * B * 4,), _F32),
            pltpu.VMEM((B, 8), _F32),
            pltpu.VMEM((B * 8,), _F32),
            pltpu.SemaphoreType.DMA,
            pltpu.SemaphoreType.DMA,
            pltpu.SemaphoreType.DMA,
            pltpu.SemaphoreType.DMA,
        ],
        compiler_params=pltpu.CompilerParams(needs_layout_passes=False),
    )
    return k(s, vt, vtail, batch)


VSUB = 128


def _vapply_body(vt_hbm, pf_hbm, b_hbm, out_hbm,
                 vb0, vb1, vb2, bv, pv, acc2, winv_v,
                 sr0, sr1, sr2, sw0, sw1, sw2, sem_p, sem_b):
    wid = lax.axis_index("s") * 2 + lax.axis_index("c")
    iota16 = lax.iota(_I32, 16)

    @pl.when(wid < NFULL)
    def _():
        base = wid * CHV
        vbs = (vb0, vb1, vb2)
        srs = (sr0, sr1, sr2)
        sws = (sw0, sw1, sw2)
        hb = pltpu.async_copy(b_hbm.at[pl.ds(base, CHV)], bv, sem_b)
        hp = pltpu.async_copy(pf_hbm, pv, sem_p)

        def _rd(k):
            return pltpu.async_copy(
                vt_hbm.at[pl.ds(0, 3), pl.ds(0, VDIM),
                          pl.ds(base + k * VSUB, VSUB)], vbs[k % 3],
                srs[k % 3])

        def _wr(k):
            return pltpu.async_copy(
                vbs[k % 3],
                out_hbm.at[pl.ds(0, 3), pl.ds(0, VDIM),
                           pl.ds(base + k * VSUB, VSUB)], sws[k % 3])

        hr = [None, None, None]
        hw = [None, None, None]
        hr[0] = _rd(0)
        hr[1] = _rd(1)
        hp.wait()
        for j in range(B * 8 // 16):
            tot = jnp.zeros((16,), _F32)
            for t in range(NW):
                tot = tot + pv[t, pl.ds(j * 16, 16)]
            acc2[pl.ds(j * 16, 16)] = tot
        for g in range(4):
            segv = g * 16 + iota16
            cnt = plsc.load_gather(acc2, [segv * 8])
            wsum = plsc.load_gather(acc2, [segv * 8 + 3])
            wm = jnp.maximum(wsum / jnp.maximum(cnt, 1.0), EPS)
            winv_v[pl.ds(g * 16, 16)] = 1.0 / wm
        hb.wait()

        for k in range(CHV // VSUB):
            b3 = k % 3
            hr[b3].wait()
            facs = []
            for g in range(8):
                segs = bv[pl.ds(k * VSUB + g * 16, 16)]
                segs = jnp.minimum(jnp.maximum(segs, 0), B - 1)
                facs.append(plsc.load_gather(winv_v, [segs]))
            vb = vbs[b3]

            def _row(j2, c, _vb=vb, _facs=facs):
                for kk in range(3):
                    for g in range(8):
                        x = _vb[kk, j2, pl.ds(g * 16, 16)]
                        _vb[kk, j2, pl.ds(g * 16, 16)] = x * _facs[g]
                return c

            lax.fori_loop(0, VDIM, _row, 0)
            hw[b3] = _wr(k)
            nk = k + 2
            if nk < CHV // VSUB:
                bb = nk % 3
                if hw[bb] is not None:
                    hw[bb].wait()
                hr[bb] = _rd(nk)
        for b3 in range(3):
            if hw[b3] is not None:
                hw[b3].wait()


def _vapply_call(vt, pflat, batch):
    mesh = plsc.VectorSubcoreMesh(core_axis_name="c", subcore_axis_name="s")
    k = pl.kernel(
        _vapply_body,
        mesh=mesh,
        out_type=jax.ShapeDtypeStruct((3, VDIM, N), _F32),
        scratch_types=[
            pltpu.VMEM((3, VDIM, VSUB), _F32),
            pltpu.VMEM((3, VDIM, VSUB), _F32),
            pltpu.VMEM((3, VDIM, VSUB), _F32),
            pltpu.VMEM((CHV,), _I32),
            pltpu.VMEM((NW, B * 8), _F32),
            pltpu.VMEM((B * 8,), _F32),
            pltpu.VMEM((B,), _F32),
            pltpu.SemaphoreType.DMA,
            pltpu.SemaphoreType.DMA,
            pltpu.SemaphoreType.DMA,
            pltpu.SemaphoreType.DMA,
            pltpu.SemaphoreType.DMA,
            pltpu.SemaphoreType.DMA,
            pltpu.SemaphoreType.DMA,
            pltpu.SemaphoreType.DMA,
        ],
        compiler_params=pltpu.CompilerParams(needs_layout_passes=False),
    )
    return k(vt, pflat, batch)


def _finalize_tbl(p_ref):
    tbl = p_ref[0:B, :]
    for t in range(1, NW):
        tbl = tbl + p_ref[t * B:(t + 1) * B, :]
    cnt = jnp.maximum(tbl[:, 0:1], 1.0)
    m = tbl[:, 1:2] / cnt
    ex2 = tbl[:, 2:3] / cnt
    var = jnp.maximum(ex2 - m * m, EPS)
    prec = lax.rsqrt(var)
    wm = jnp.maximum(tbl[:, 3:4] / cnt, EPS)
    winv = 1.0 / wm
    return m, prec, winv


def _apply_body(s_ref, v_ref, bc_ref, br_ref, p_ref, w_ref, bb_ref,
                so_ref, vo_ref, st_ref):
    @pl.when(pl.program_id(0) == 0)
    def _():
        m, prec, winv = _finalize_tbl(p_ref)
        col = lax.broadcasted_iota(_I32, (B, 8), 1)
        st_ref[...] = jnp.where(
            col == 0, m, jnp.where(col == 1, prec,
                                   jnp.where(col == 2, winv, 0.0)))

    stats = st_ref[...]
    bc = bc_ref[...]
    nb = bc.shape[0]
    oh = (bc == lax.broadcasted_iota(_I32, (nb, B), 1)).astype(_F32)
    g = jnp.dot(oh, stats, preferred_element_type=_F32)
    mg = g[:, 0:1]
    pg = g[:, 1:2]
    so_ref[...] = (s_ref[...] - mg) * pg * w_ref[...] + bb_ref[...]

    br = br_ref[...]
    ohv = (br == lax.broadcasted_iota(_I32, (B, 128), 0)).astype(_F32)
    wrow = jnp.sum(stats[:, 2:3] * ohv, axis=0, keepdims=True)
    vo_ref[...] = v_ref[...] * wrow[None]


def _apply(s, vt, bcol, brow, p2, w2, b2, nb):
    grid = ((N + nb - 1) // nb,)
    vtb = TAIL0 // 128
    return pl.pallas_call(
        _apply_body,
        grid=grid,
        in_specs=[
            pl.BlockSpec((nb, SDIM), lambda i: (i, 0)),
            pl.BlockSpec((3, VDIM, 128), lambda i: (0, 0, vtb)),
            pl.BlockSpec((nb, 1), lambda i: (i, 0)),
            pl.BlockSpec((1, 128), lambda i: (0, vtb)),
            pl.BlockSpec((NW * B, 8), lambda i: (0, 0)),
            pl.BlockSpec((1, SDIM), lambda i: (0, 0)),
            pl.BlockSpec((1, SDIM), lambda i: (0, 0)),
        ],
        out_specs=[
            pl.BlockSpec((nb, SDIM), lambda i: (i, 0)),
            pl.BlockSpec((3, VDIM, 128), lambda i: (0, 0, 0)),
        ],
        out_shape=[
            jax.ShapeDtypeStruct((N, SDIM), _F32),
            jax.ShapeDtypeStruct((3, VDIM, 128), _F32),
        ],
        scratch_shapes=[pltpu.VMEM((B, 8), _F32)],
        compiler_params=pltpu.CompilerParams(
            dimension_semantics=("arbitrary",),
            vmem_limit_bytes=100 * 1024 * 1024,
        ),
    )(s, vt, bcol, brow, p2, w2, b2)


def kernel(s, v, batch, weight, bias):
    vt = jnp.transpose(v, (2, 1, 0))
    vtail = vt[:, 0, TAIL0:]
    batch = batch.astype(_I32)
    partials, brow, pflat = _stats_call(s, vt, vtail, batch)
    p2 = partials.reshape(NW * B, 8)
    voutT_sc = _vapply_call(vt, pflat, batch)
    sout, vtail_out = _apply(s, vt, batch.reshape(N, 1), brow, p2,
                             weight.reshape(1, SDIM), bias.reshape(1, SDIM),
                             6400)
    voutT = lax.dynamic_update_slice(
        voutT_sc, vtail_out[:, :, :TAILN], (0, 0, TAIL0))
    return sout, jnp.transpose(voutT, (2, 1, 0))


# speedup vs baseline: 1.1653x; 1.1653x over previous
import jax
import jax.numpy as jnp
from jax import lax
from jax.experimental import pallas as pl
from jax.experimental.pallas import tpu as pltpu
from jax.experimental.pallas import tpu_sc as plsc

N = 50000
SDIM = 256
VDIM = 64
B = 64
EPS = 1e-6

NW = 32
CHV = 1664
NFULL = 30
TAIL0 = NFULL * CHV
TAILN = N - TAIL0
SSUB = 208
NSSUB = CHV // SSUB

_F32 = jnp.float32
_I32 = jnp.int32


NPAD = 50048


def _stats_body(s_hbm, vt_hbm, vtail_hbm, b_hbm, out_hbm, brow_hbm,
                sv0, sv1, vv, vtv, bv, acc, loc, sem_s0, sem_s1, sem_v,
                sem_b):
    wid = lax.axis_index("s") * 2 + lax.axis_index("c")
    iota16 = lax.iota(_I32, 16)
    zeros16 = jnp.zeros((16,), _F32)
    lane0 = iota16 == 0
    lanebase = iota16 * 64

    def _z(j, c):
        acc[pl.ds(j * 16, 16)] = zeros16
        return c

    lax.fori_loop(0, 256, _z, 0)

    def _clamp(segs):
        return jnp.minimum(jnp.maximum(segs, 0), B - 1)

    def _vgroup(goff):
        segs = _clamp(bv[pl.ds(goff * 16, 16)])
        v0 = vv[0, 0, pl.ds(goff * 16, 16)]
        v1 = vv[1, 0, pl.ds(goff * 16, 16)]
        v2 = vv[2, 0, pl.ds(goff * 16, 16)]
        vsq = v0 * v0 + v1 * v1 + v2 * v2
        plsc.addupdate_scatter(acc, [lanebase + segs + 3072], vsq)

    def _sgroup(sv, boff, row0):
        segs = _clamp(bv[pl.ds(boff, 16)])
        umax = lax.reduce_max(segs, (0,))
        umin = lax.reduce_min(segs, (0,))

        @pl.when(umax == umin)
        def _():
            cs = zeros16
            sq = zeros16
            for jj in range(16):
                x = sv[row0 + jj, pl.ds(0, 16)]
                cs = cs + x
                sq = sq + x * x
            idx = jnp.full((16,), umax, _I32)
            plsc.addupdate_scatter(acc, [idx], jnp.full((16,), 16.0, _F32),
                                   mask=lane0)
            plsc.addupdate_scatter(acc, [idx + 1024], cs, mask=lane0)
            plsc.addupdate_scatter(acc, [idx + 2048], sq, mask=lane0)

        @pl.when(umax != umin)
        def _():
            for jj in range(16):
                x = sv[row0 + jj, pl.ds(0, 16)]
                idx = jnp.full((16,), segs[jj], _I32)
                plsc.addupdate_scatter(acc, [idx], jnp.full((16,), 1.0, _F32),
                                       mask=lane0)
                plsc.addupdate_scatter(acc, [idx + 1024], x, mask=lane0)
                plsc.addupdate_scatter(acc, [idx + 2048], x * x, mask=lane0)

    @pl.when(wid < NFULL)
    def _():
        base = wid * CHV
        svs = (sv0, sv1)
        sems = (sem_s0, sem_s1)
        hb = pltpu.async_copy(b_hbm.at[pl.ds(base, CHV)], bv, sem_b)
        hv = pltpu.async_copy(
            vt_hbm.at[pl.ds(0, 3), pl.ds(0, 8), pl.ds(base, CHV)], vv, sem_v)

        def _start_s(k):
            return pltpu.async_copy(
                s_hbm.at[pl.ds(base + k * SSUB, SSUB), pl.ds(0, 128)],
                svs[k % 2], sems[k % 2])

        hs = _start_s(0)
        hb.wait()
        pltpu.sync_copy(bv, brow_hbm.at[0, pl.ds(base, CHV)])
        for k in range(NSSUB):
            hs_next = _start_s(k + 1) if k + 1 < NSSUB else None
            hs.wait()

            def _sg(j, c2, _k=k):
                _sgroup(svs[_k % 2], _k * SSUB + j * 16, j * 16)
                return c2

            lax.fori_loop(0, SSUB // 16, _sg, 0)
            hs = hs_next

        hv.wait()

        def _vg(g, c):
            _vgroup(g)
            return c

        lax.fori_loop(0, CHV // 16, _vg, 0)

    @pl.when(wid == NFULL)
    def _():
        pltpu.sync_copy(b_hbm.at[pl.ds(TAIL0, TAILN)], bv.at[pl.ds(0, TAILN)])
        pltpu.sync_copy(bv.at[pl.ds(0, 128)],
                        brow_hbm.at[0, pl.ds(TAIL0, 128)])
        pltpu.sync_copy(s_hbm.at[pl.ds(TAIL0, TAILN), pl.ds(0, 128)],
                        sv0.at[pl.ds(0, TAILN)])
        for j in range(TAILN // 16):
            _sgroup(sv0, j * 16, j * 16)

    @pl.when(wid == NFULL + 1)
    def _():
        pltpu.sync_copy(b_hbm.at[pl.ds(TAIL0, TAILN)], bv.at[pl.ds(0, TAILN)])
        pltpu.sync_copy(vtail_hbm, vtv)
        for g in range(TAILN // 16):
            segs = _clamp(bv[pl.ds(g * 16, 16)])
            v0 = vtv[0, pl.ds(g * 16, 16)]
            v1 = vtv[1, pl.ds(g * 16, 16)]
            v2 = vtv[2, pl.ds(g * 16, 16)]
            vsq = v0 * v0 + v1 * v1 + v2 * v2
            plsc.addupdate_scatter(acc, [lanebase + segs + 3072], vsq)

    for g in range(4):
        segv = g * 16 + iota16
        for f in range(4, 8):
            plsc.store_scatter(loc, [segv, jnp.full((16,), f, _I32)], zeros16)
        for f in range(4):
            tot = zeros16
            for l in range(16):
                tot = tot + acc[pl.ds(f * 1024 + l * 64 + g * 16, 16)]
            plsc.store_scatter(loc, [segv, jnp.full((16,), f, _I32)], tot)
    pltpu.sync_copy(loc, out_hbm.at[wid])


def _stats_call(s, vt, vtail, batch):
    mesh = plsc.VectorSubcoreMesh(core_axis_name="c", subcore_axis_name="s")
    k = pl.kernel(
        _stats_body,
        mesh=mesh,
        out_type=(jax.ShapeDtypeStruct((NW, B, 8), _F32),
                  jax.ShapeDtypeStruct((1, NPAD), _I32)),
        scratch_types=[
            pltpu.VMEM((SSUB, 128), _F32),
            pltpu.VMEM((SSUB, 128), _F32),
            pltpu.VMEM((3, 8, CHV), _F32),
            pltpu.VMEM((3, TAILN), _F32),
            pltpu.VMEM((CHV,), _I32),
            pltpu.VMEM((16 * B * 4,), _F32),
            pltpu.VMEM((B, 8), _F32),
            pltpu.SemaphoreType.DMA,
            pltpu.SemaphoreType.DMA,
            pltpu.SemaphoreType.DMA,
            pltpu.SemaphoreType.DMA,
        ],
        compiler_params=pltpu.CompilerParams(needs_layout_passes=False),
    )
    return k(s, vt, vtail, batch)


def _finalize_tbl(p_ref):
    tbl = p_ref[0:B, :]
    for t in range(1, NW):
        tbl = tbl + p_ref[t * B:(t + 1) * B, :]
    cnt = jnp.maximum(tbl[:, 0:1], 1.0)
    m = tbl[:, 1:2] / cnt
    ex2 = tbl[:, 2:3] / cnt
    var = jnp.maximum(ex2 - m * m, EPS)
    prec = lax.rsqrt(var)
    wm = jnp.maximum(tbl[:, 3:4] / cnt, EPS)
    winv = 1.0 / wm
    return m, prec, winv


def _apply_body(s_ref, v_ref, bc_ref, br_ref, p_ref, w_ref, bb_ref,
                so_ref, vo_ref, st_ref):
    @pl.when(pl.program_id(0) == 0)
    def _():
        m, prec, winv = _finalize_tbl(p_ref)
        col = lax.broadcasted_iota(_I32, (B, 8), 1)
        st_ref[...] = jnp.where(
            col == 0, m, jnp.where(col == 1, prec,
                                   jnp.where(col == 2, winv, 0.0)))

    stats = st_ref[...]
    bc = bc_ref[...]
    nb = bc.shape[0]
    oh = (bc == lax.broadcasted_iota(_I32, (nb, B), 1)).astype(_F32)
    g = jnp.dot(oh, stats, preferred_element_type=_F32)
    mg = g[:, 0:1]
    pg = g[:, 1:2]
    so_ref[...] = (s_ref[...] - mg) * pg * w_ref[...] + bb_ref[...]

    br = br_ref[...]
    ohv = (br == lax.broadcasted_iota(_I32, (B, nb), 0)).astype(_F32)
    wrow = jnp.sum(stats[:, 2:3] * ohv, axis=0, keepdims=True)
    vo_ref[...] = v_ref[...] * wrow[None]


def _apply(s, vt, bcol, brow, p2, w2, b2, nb):
    grid = ((N + nb - 1) // nb,)
    return pl.pallas_call(
        _apply_body,
        grid=grid,
        in_specs=[
            pl.BlockSpec((nb, SDIM), lambda i: (i, 0)),
            pl.BlockSpec((3, VDIM, nb), lambda i: (0, 0, i)),
            pl.BlockSpec((nb, 1), lambda i: (i, 0)),
            pl.BlockSpec((1, nb), lambda i: (0, i)),
            pl.BlockSpec((NW * B, 8), lambda i: (0, 0)),
            pl.BlockSpec((1, SDIM), lambda i: (0, 0)),
            pl.BlockSpec((1, SDIM), lambda i: (0, 0)),
        ],
        out_specs=[
            pl.BlockSpec((nb, SDIM), lambda i: (i, 0)),
            pl.BlockSpec((3, VDIM, nb), lambda i: (0, 0, i)),
        ],
        out_shape=[
            jax.ShapeDtypeStruct((N, SDIM), _F32),
            jax.ShapeDtypeStruct((3, VDIM, N), _F32),
        ],
        scratch_shapes=[pltpu.VMEM((B, 8), _F32)],
        compiler_params=pltpu.CompilerParams(
            dimension_semantics=("arbitrary",),
            vmem_limit_bytes=100 * 1024 * 1024,
        ),
    )(s, vt, bcol, brow, p2, w2, b2)


def kernel(s, v, batch, weight, bias):
    vt = jnp.transpose(v, (2, 1, 0))
    vtail = vt[:, 0, TAIL0:]
    batch = batch.astype(_I32)
    partials, brow = _stats_call(s, vt, vtail, batch)
    p2 = partials.reshape(NW * B, 8)
    sout, voutT = _apply(s, vt, batch.reshape(N, 1), brow, p2,
                         weight.reshape(1, SDIM), bias.reshape(1, SDIM), 6400)
    return sout, jnp.transpose(voutT, (2, 1, 0))
